# baseline (speedup 1.0000x reference)
import functools

import jax
import jax.numpy as jnp
from jax import lax
from jax.experimental import pallas as pl
from jax.experimental.pallas import tpu as pltpu
from jax.experimental.pallas import tpu_sc as plsc

N = 10000
E = 320000
D = 128

NC = 2
NS = 16
T = NC * NS
C = 64
CHUNKS = 160
EPT = C * CHUNKS
NPAD = 10112
RPT = NPAD // NS
DEGW = 128

_mesh = plsc.VectorSubcoreMesh(core_axis_name="c", subcore_axis_name="s")


@functools.partial(
    pl.kernel,
    out_type=jax.ShapeDtypeStruct((NC, NPAD, DEGW), jnp.float32),
    mesh=_mesh,
    scratch_types=[
        pltpu.VMEM((CHUNKS, C), jnp.int32),
        pltpu.VMEM((C, DEGW), jnp.float32),
        pltpu.VMEM_SHARED((NPAD, DEGW), jnp.float32),
    ],
)
def _sc_deg(dst_hbm, ones_hbm, zeros_hbm, out_hbm, dst_v, ones_v, acc):
    c = lax.axis_index("c")
    s = lax.axis_index("s")
    w = s * NC + c
    pltpu.sync_copy(zeros_hbm, acc.at[pl.ds(s * RPT, RPT)])
    pltpu.sync_copy(ones_hbm, ones_v)
    pltpu.sync_copy(dst_hbm.at[w], dst_v)
    plsc.subcore_barrier()

    def body(j, carry):
        pltpu.sync_copy(ones_v, acc.at[dst_v.at[j]], add=True)
        return carry

    lax.fori_loop(0, CHUNKS, body, 0)
    plsc.subcore_barrier()
    pltpu.sync_copy(acc.at[pl.ds(s * RPT, RPT)],
                    out_hbm.at[c, pl.ds(s * RPT, RPT)])


IB = 8
NB = CHUNKS // IB
NBUF = 4


@functools.partial(
    pl.kernel,
    out_type=jax.ShapeDtypeStruct((NC, NPAD, D), jnp.float32),
    mesh=_mesh,
    scratch_types=[
        pltpu.VMEM((IB, C), jnp.int32),
        pltpu.VMEM((IB, C), jnp.int32),
        pltpu.VMEM((C, D), jnp.float32),
        pltpu.VMEM((C, D), jnp.float32),
        pltpu.VMEM((C, D), jnp.float32),
        pltpu.VMEM((C, D), jnp.float32),
        pltpu.VMEM_SHARED((NPAD, D), jnp.float32),
        pltpu.SemaphoreType.DMA,
        pltpu.SemaphoreType.DMA,
        pltpu.SemaphoreType.DMA,
        pltpu.SemaphoreType.DMA,
    ],
)
def _sc_prop(g_hbm, src_hbm, dst_hbm, zeros_hbm, out_hbm,
             src_v, dst_v, rows0, rows1, rows2, rows3, acc,
             sem0, sem1, sem2, sem3):
    c = lax.axis_index("c")
    s = lax.axis_index("s")
    w = s * NC + c
    pltpu.sync_copy(zeros_hbm, acc.at[pl.ds(s * RPT, RPT)])
    plsc.subcore_barrier()

    rows = (rows0, rows1, rows2, rows3)
    sems = (sem0, sem1, sem2, sem3)

    def body(blk, carry):
        pltpu.sync_copy(src_hbm.at[w, pl.ds(blk * IB, IB)], src_v)
        pltpu.sync_copy(dst_hbm.at[w, pl.ds(blk * IB, IB)], dst_v)
        descs = [None] * NBUF
        for p in range(NBUF - 1):
            descs[p] = pltpu.async_copy(
                g_hbm.at[src_v.at[p]], rows[p], sems[p])
        for b in range(IB):
            nxt = b + NBUF - 1
            if nxt < IB:
                descs[nxt % NBUF] = pltpu.async_copy(
                    g_hbm.at[src_v.at[nxt]], rows[nxt % NBUF],
                    sems[nxt % NBUF])
            descs[b % NBUF].wait()
            pltpu.sync_copy(rows[b % NBUF], acc.at[dst_v.at[b]], add=True)
        return carry

    lax.fori_loop(0, NB, body, 0)

    plsc.subcore_barrier()
    pltpu.sync_copy(acc.at[pl.ds(s * RPT, RPT)],
                    out_hbm.at[c, pl.ds(s * RPT, RPT)])


_BN = 1000


def _tc1_body(dp_ref, x_ref, g0_ref, nb_ref):
    dp = dp_ref[...]
    deg = dp[0, :, 0:1] + dp[1, :, 0:1]
    norm = lax.rsqrt(jnp.maximum(deg, 1.0))
    nb = jnp.broadcast_to(norm, x_ref.shape)
    g0_ref[...] = x_ref[...] * nb
    nb_ref[...] = nb


def _tc1(degp, x):
    grid = (N // _BN,)
    return pl.pallas_call(
        _tc1_body,
        grid=grid,
        in_specs=[
            pl.BlockSpec((NC, _BN, DEGW), lambda i: (0, i, 0)),
            pl.BlockSpec((_BN, D), lambda i: (i, 0)),
        ],
        out_specs=[
            pl.BlockSpec((_BN, D), lambda i: (i, 0)),
            pl.BlockSpec((_BN, D), lambda i: (i, 0)),
        ],
        out_shape=[
            jax.ShapeDtypeStruct((N, D), jnp.float32),
            jax.ShapeDtypeStruct((N, D), jnp.float32),
        ],
    )(degp, x)


def _tc2_body(p_ref, nb_ref, f1_ref, g1_ref):
    ssum = p_ref[0] + p_ref[1]
    nb = nb_ref[...]
    f1 = ssum * nb
    f1_ref[...] = f1
    g1_ref[...] = f1 * nb


def _tc2(p1, nb):
    grid = (N // _BN,)
    return pl.pallas_call(
        _tc2_body,
        grid=grid,
        in_specs=[
            pl.BlockSpec((NC, _BN, D), lambda i: (0, i, 0)),
            pl.BlockSpec((_BN, D), lambda i: (i, 0)),
        ],
        out_specs=[
            pl.BlockSpec((_BN, D), lambda i: (i, 0)),
            pl.BlockSpec((_BN, D), lambda i: (i, 0)),
        ],
        out_shape=[
            jax.ShapeDtypeStruct((N, D), jnp.float32),
            jax.ShapeDtypeStruct((N, D), jnp.float32),
        ],
    )(p1, nb)


def _tc3_body(p_ref, nb_ref, x_ref, f1_ref, w_ref, b_ref, a_ref, o_ref):
    f2 = (p_ref[0] + p_ref[1]) * nb_ref[...]
    s = a_ref[0] * x_ref[...] + a_ref[1] * f1_ref[...] + a_ref[2] * f2
    o_ref[...] = lax.dot_general(
        s, w_ref[...], (((1,), (1,)), ((), ())),
        preferred_element_type=jnp.float32) + 3.0 * b_ref[...]


def _tc3(p2, nb, x, f1, W, b2, alpha):
    grid = (N // _BN,)
    return pl.pallas_call(
        _tc3_body,
        grid=grid,
        in_specs=[
            pl.BlockSpec((NC, _BN, D), lambda i: (0, i, 0)),
            pl.BlockSpec((_BN, D), lambda i: (i, 0)),
            pl.BlockSpec((_BN, D), lambda i: (i, 0)),
            pl.BlockSpec((_BN, D), lambda i: (i, 0)),
            pl.BlockSpec((D, D), lambda i: (0, 0)),
            pl.BlockSpec((1, D), lambda i: (0, 0)),
            pl.BlockSpec(memory_space=pltpu.SMEM),
        ],
        out_specs=pl.BlockSpec((_BN, D), lambda i: (i, 0)),
        out_shape=jax.ShapeDtypeStruct((N, D), jnp.float32),
    )(p2, nb, x, f1, W, b2, alpha)


def kernel(x, edge_index, W, b, alpha):
    src = edge_index[0]
    dst = edge_index[1]
    pad = T * EPT - E
    srcp = jnp.concatenate(
        [src, jnp.zeros((pad,), jnp.int32)]).reshape(T, CHUNKS, C)
    dstp = jnp.concatenate(
        [dst, jnp.full((pad,), N, jnp.int32)]).reshape(T, CHUNKS, C)
    ones_rows = jnp.ones((C, DEGW), jnp.float32)
    z16 = jnp.zeros((RPT, DEGW), jnp.float32)
    z128 = jnp.zeros((RPT, D), jnp.float32)

    degp = _sc_deg(dstp, ones_rows, z16)[:, :N, :]
    g0, nb = _tc1(degp, x)
    p1 = _sc_prop(g0, srcp, dstp, z128)[:, :N, :]
    f1, g1 = _tc2(p1, nb)
    p2 = _sc_prop(g1, srcp, dstp, z128)[:, :N, :]
    return _tc3(p2, nb, x, f1, W, b.reshape(1, D), alpha)


# speedup vs baseline: 1.0206x; 1.0206x over previous
import functools

import jax
import jax.numpy as jnp
from jax import lax
from jax.experimental import pallas as pl
from jax.experimental.pallas import tpu as pltpu
from jax.experimental.pallas import tpu_sc as plsc

N = 10000
E = 320000
D = 128

NC = 2
NS = 16
T = NC * NS
C = 128
CHUNKS = 80
EPT = C * CHUNKS
NPAD = 10112
RPT = NPAD // NS
DEGW = 128

_mesh = plsc.VectorSubcoreMesh(core_axis_name="c", subcore_axis_name="s")


@functools.partial(
    pl.kernel,
    out_type=jax.ShapeDtypeStruct((NC, NPAD, DEGW), jnp.float32),
    mesh=_mesh,
    scratch_types=[
        pltpu.VMEM((CHUNKS, C), jnp.int32),
        pltpu.VMEM((C, DEGW), jnp.float32),
        pltpu.VMEM_SHARED((NPAD, DEGW), jnp.float32),
    ],
)
def _sc_deg(dst_hbm, ones_hbm, zeros_hbm, out_hbm, dst_v, ones_v, acc):
    c = lax.axis_index("c")
    s = lax.axis_index("s")
    w = s * NC + c
    pltpu.sync_copy(zeros_hbm, acc.at[pl.ds(s * RPT, RPT)])
    pltpu.sync_copy(ones_hbm, ones_v)
    pltpu.sync_copy(dst_hbm.at[w], dst_v)
    plsc.subcore_barrier()

    def body(j, carry):
        pltpu.sync_copy(ones_v, acc.at[dst_v.at[j]], add=True)
        return carry

    lax.fori_loop(0, CHUNKS, body, 0)
    plsc.subcore_barrier()
    pltpu.sync_copy(acc.at[pl.ds(s * RPT, RPT)],
                    out_hbm.at[c, pl.ds(s * RPT, RPT)])


IB = 8
NB = CHUNKS // IB


@functools.partial(
    pl.kernel,
    out_type=jax.ShapeDtypeStruct((NC, NPAD, D), jnp.float32),
    mesh=_mesh,
    scratch_types=[
        pltpu.VMEM((IB, C), jnp.int32),
        pltpu.VMEM((IB, C), jnp.int32),
        pltpu.VMEM((C, D), jnp.float32),
        pltpu.VMEM((C, D), jnp.float32),
        pltpu.VMEM_SHARED((NPAD, D), jnp.float32),
        pltpu.SemaphoreType.DMA,
        pltpu.SemaphoreType.DMA,
    ],
)
def _sc_prop(g_hbm, src_hbm, dst_hbm, zeros_hbm, out_hbm,
             src_v, dst_v, rows0, rows1, acc, sem0, sem1):
    c = lax.axis_index("c")
    s = lax.axis_index("s")
    w = s * NC + c
    pltpu.sync_copy(zeros_hbm, acc.at[pl.ds(s * RPT, RPT)])
    plsc.subcore_barrier()

    rows = (rows0, rows1)
    sems = (sem0, sem1)

    def body(blk, carry):
        pltpu.sync_copy(src_hbm.at[w, pl.ds(blk * IB, IB)], src_v)
        pltpu.sync_copy(dst_hbm.at[w, pl.ds(blk * IB, IB)], dst_v)
        descs = [pltpu.async_copy(g_hbm.at[src_v.at[0]], rows0, sem0), None]
        for b in range(IB):
            nxt = b + 1
            if nxt < IB:
                descs[nxt % 2] = pltpu.async_copy(
                    g_hbm.at[src_v.at[nxt]], rows[nxt % 2], sems[nxt % 2])
            descs[b % 2].wait()
            pltpu.sync_copy(rows[b % 2], acc.at[dst_v.at[b]], add=True)
        return carry

    lax.fori_loop(0, NB, body, 0)

    plsc.subcore_barrier()
    pltpu.sync_copy(acc.at[pl.ds(s * RPT, RPT)],
                    out_hbm.at[c, pl.ds(s * RPT, RPT)])


_BN = 1000


def _tc1_body(dp_ref, x_ref, g0_ref, nb_ref):
    dp = dp_ref[...]
    deg = dp[0, :, 0:1] + dp[1, :, 0:1]
    norm = lax.rsqrt(jnp.maximum(deg, 1.0))
    nb = jnp.broadcast_to(norm, x_ref.shape)
    g0_ref[...] = x_ref[...] * nb
    nb_ref[...] = nb


def _tc1(degp, x):
    grid = (N // _BN,)
    return pl.pallas_call(
        _tc1_body,
        grid=grid,
        in_specs=[
            pl.BlockSpec((NC, _BN, DEGW), lambda i: (0, i, 0)),
            pl.BlockSpec((_BN, D), lambda i: (i, 0)),
        ],
        out_specs=[
            pl.BlockSpec((_BN, D), lambda i: (i, 0)),
            pl.BlockSpec((_BN, D), lambda i: (i, 0)),
        ],
        out_shape=[
            jax.ShapeDtypeStruct((N, D), jnp.float32),
            jax.ShapeDtypeStruct((N, D), jnp.float32),
        ],
    )(degp, x)


def _tc2_body(p_ref, nb_ref, f1_ref, g1_ref):
    ssum = p_ref[0] + p_ref[1]
    nb = nb_ref[...]
    f1 = ssum * nb
    f1_ref[...] = f1
    g1_ref[...] = f1 * nb


def _tc2(p1, nb):
    grid = (N // _BN,)
    return pl.pallas_call(
        _tc2_body,
        grid=grid,
        in_specs=[
            pl.BlockSpec((NC, _BN, D), lambda i: (0, i, 0)),
            pl.BlockSpec((_BN, D), lambda i: (i, 0)),
        ],
        out_specs=[
            pl.BlockSpec((_BN, D), lambda i: (i, 0)),
            pl.BlockSpec((_BN, D), lambda i: (i, 0)),
        ],
        out_shape=[
            jax.ShapeDtypeStruct((N, D), jnp.float32),
            jax.ShapeDtypeStruct((N, D), jnp.float32),
        ],
    )(p1, nb)


def _tc3_body(p_ref, nb_ref, x_ref, f1_ref, w_ref, b_ref, a_ref, o_ref):
    f2 = (p_ref[0] + p_ref[1]) * nb_ref[...]
    s = a_ref[0] * x_ref[...] + a_ref[1] * f1_ref[...] + a_ref[2] * f2
    o_ref[...] = lax.dot_general(
        s, w_ref[...], (((1,), (1,)), ((), ())),
        preferred_element_type=jnp.float32) + 3.0 * b_ref[...]


def _tc3(p2, nb, x, f1, W, b2, alpha):
    grid = (N // _BN,)
    return pl.pallas_call(
        _tc3_body,
        grid=grid,
        in_specs=[
            pl.BlockSpec((NC, _BN, D), lambda i: (0, i, 0)),
            pl.BlockSpec((_BN, D), lambda i: (i, 0)),
            pl.BlockSpec((_BN, D), lambda i: (i, 0)),
            pl.BlockSpec((_BN, D), lambda i: (i, 0)),
            pl.BlockSpec((D, D), lambda i: (0, 0)),
            pl.BlockSpec((1, D), lambda i: (0, 0)),
            pl.BlockSpec(memory_space=pltpu.SMEM),
        ],
        out_specs=pl.BlockSpec((_BN, D), lambda i: (i, 0)),
        out_shape=jax.ShapeDtypeStruct((N, D), jnp.float32),
    )(p2, nb, x, f1, W, b2, alpha)


def kernel(x, edge_index, W, b, alpha):
    src = edge_index[0]
    dst = edge_index[1]
    pad = T * EPT - E
    srcp = jnp.concatenate(
        [src, jnp.zeros((pad,), jnp.int32)]).reshape(T, CHUNKS, C)
    dstp = jnp.concatenate(
        [dst, jnp.full((pad,), N, jnp.int32)]).reshape(T, CHUNKS, C)
    ones_rows = jnp.ones((C, DEGW), jnp.float32)
    z16 = jnp.zeros((RPT, DEGW), jnp.float32)
    z128 = jnp.zeros((RPT, D), jnp.float32)

    degp = _sc_deg(dstp, ones_rows, z16)[:, :N, :]
    g0, nb = _tc1(degp, x)
    p1 = _sc_prop(g0, srcp, dstp, z128)[:, :N, :]
    f1, g1 = _tc2(p1, nb)
    p2 = _sc_prop(g1, srcp, dstp, z128)[:, :N, :]
    return _tc3(p2, nb, x, f1, W, b.reshape(1, D), alpha)
